# baseline (speedup 1.0000x reference)
import functools

import jax
import jax.numpy as jnp
from jax import lax
from jax.experimental import pallas as pl
from jax.experimental.pallas import tpu as pltpu
from jax.experimental.pallas import tpu_sc as plsc

B = 16384
V_S0 = 100000
V_S1 = 1000
V_S2 = 1000
V_S3 = 1000000
V_CR = V_S1 * V_S2
NC = 2
NS = 16
NW = NC * NS
BPW = B // NW
NCHUNK = 4
CHUNK = BPW // NCHUNK


def _gather_body(d0, d1, d2, d3, s0, s1, s2, s3, fw, f0, f1, f2, f3, fc,
                 out,
                 idx0, idx3, cidx, idx1_v, idx2_v,
                 g0_v, g3_v, gc_v, t1_v, t2_v,
                 dv0, dv1, dv2, dv3, w_v, out_v,
                 sem0, sem3, semc, semi, semj, semm):
    wid = lax.axis_index("s") * NC + lax.axis_index("c")
    base = wid * BPW

    cp_idx = []
    for j in range(NCHUNK):
        cp_idx.append(pltpu.async_copy(
            s0.at[pl.ds(base + j * CHUNK, CHUNK)], idx0.at[j], semi))
        cp_idx.append(pltpu.async_copy(
            s3.at[pl.ds(base + j * CHUNK, CHUNK)], idx3.at[j], semi))
    cp_i1 = pltpu.async_copy(s1.at[pl.ds(base, BPW)], idx1_v, semj)
    cp_i2 = pltpu.async_copy(s2.at[pl.ds(base, BPW)], idx2_v, semj)
    cp_m = [
        pltpu.async_copy(f1, t1_v, semm),
        pltpu.async_copy(f2, t2_v, semm),
        pltpu.async_copy(d0.at[pl.ds(base, BPW)], dv0, semm),
        pltpu.async_copy(d1.at[pl.ds(base, BPW)], dv1, semm),
        pltpu.async_copy(d2.at[pl.ds(base, BPW)], dv2, semm),
        pltpu.async_copy(d3.at[pl.ds(base, BPW)], dv3, semm),
        pltpu.async_copy(fw, w_v, semm),
    ]

    for cp in cp_idx:
        cp.wait()
    cps = []
    for j in range(NCHUNK):
        cps.append(pltpu.async_copy(
            f0.at[idx0.at[j]], g0_v.at[pl.ds(j * CHUNK, CHUNK)], sem0))
        cps.append(pltpu.async_copy(
            f3.at[idx3.at[j]], g3_v.at[pl.ds(j * CHUNK, CHUNK)], sem3))

    cp_i1.wait()
    cp_i2.wait()
    for j in range(NCHUNK):
        for k in range(CHUNK // 16):
            sl = pl.ds(j * CHUNK + k * 16, 16)
            a = idx1_v[sl]
            b = idx2_v[sl]
            cidx[j, pl.ds(k * 16, 16)] = a * V_S2 + b
    for j in range(NCHUNK):
        cps.append(pltpu.async_copy(
            fc.at[cidx.at[j]], gc_v.at[pl.ds(j * CHUNK, CHUNK)], semc))

    for cp in cp_m:
        cp.wait()
    w0 = w_v[pl.ds(0, 16)]
    w1 = w_v[pl.ds(16, 16)]
    w2 = w_v[pl.ds(32, 16)]
    w3 = w_v[pl.ds(48, 16)]

    for cp in cps:
        cp.wait()

    for i in range(BPW // 16):
        sl = pl.ds(i * 16, 16)
        e1 = plsc.load_gather(t1_v, [idx1_v[sl]])
        e2 = plsc.load_gather(t2_v, [idx2_v[sl]])
        acc = dv0[sl] * w0 + dv1[sl] * w1 + dv2[sl] * w2 + dv3[sl] * w3
        acc = acc + g0_v[sl] + g3_v[sl] + gc_v[sl] + e1 + e2
        out_v[sl] = acc

    pltpu.sync_copy(out_v, out.at[pl.ds(base, BPW)])


@jax.jit
def kernel(d0, d1, d2, d3, s0, s1, s2, s3, W_dense,
           emb_s0, emb_s1, emb_s2, emb_s3, emb_cross_s1_s2):
    mesh = plsc.VectorSubcoreMesh(core_axis_name="c", subcore_axis_name="s")

    gather = functools.partial(
        pl.kernel,
        mesh=mesh,
        compiler_params=pltpu.CompilerParams(needs_layout_passes=False),
        out_type=jax.ShapeDtypeStruct((B,), jnp.float32),
        scratch_types=[
            pltpu.VMEM((NCHUNK, CHUNK), jnp.int32),
            pltpu.VMEM((NCHUNK, CHUNK), jnp.int32),
            pltpu.VMEM((NCHUNK, CHUNK), jnp.int32),
            pltpu.VMEM((BPW,), jnp.int32),
            pltpu.VMEM((BPW,), jnp.int32),
            pltpu.VMEM((BPW,), jnp.float32),
            pltpu.VMEM((BPW,), jnp.float32),
            pltpu.VMEM((BPW,), jnp.float32),
            pltpu.VMEM((1024,), jnp.float32),
            pltpu.VMEM((1024,), jnp.float32),
            pltpu.VMEM((BPW,), jnp.float32),
            pltpu.VMEM((BPW,), jnp.float32),
            pltpu.VMEM((BPW,), jnp.float32),
            pltpu.VMEM((BPW,), jnp.float32),
            pltpu.VMEM((64,), jnp.float32),
            pltpu.VMEM((BPW,), jnp.float32),
            pltpu.SemaphoreType.DMA,
            pltpu.SemaphoreType.DMA,
            pltpu.SemaphoreType.DMA,
            pltpu.SemaphoreType.DMA,
            pltpu.SemaphoreType.DMA,
            pltpu.SemaphoreType.DMA,
        ],
    )(_gather_body)

    fw = jnp.broadcast_to(W_dense.reshape(4, 1), (4, 16)).reshape(64)
    pad24 = jnp.zeros((24,), jnp.float32)
    f0 = emb_s0.reshape(-1)
    f1 = jnp.concatenate([emb_s1.reshape(-1), pad24])
    f2 = jnp.concatenate([emb_s2.reshape(-1), pad24])
    f3 = emb_s3.reshape(-1)
    fc = emb_cross_s1_s2.reshape(-1)
    s0i = s0.astype(jnp.int32)
    s1i = s1.astype(jnp.int32)
    s2i = s2.astype(jnp.int32)
    s3i = s3.astype(jnp.int32)
    out = gather(d0, d1, d2, d3, s0i, s1i, s2i, s3i,
                 fw, f0, f1, f2, f3, fc)
    return out.reshape(B, 1)
